# baseline (speedup 1.0000x reference)
import functools

import jax
import jax.numpy as jnp
from jax import lax
from jax.experimental import pallas as pl
from jax.experimental.pallas import tpu as pltpu
from jax.experimental.pallas import tpu_sc as plsc

_N_MIRNA = 495
_N_DIS = 383
_N = 878
_NP = 896
_F = 256
_E = 28096
_NC = 2
_NS = 16
_SENT = _NP - 1
_W0 = 512
_W1 = _NP - _W0
_EPT = 1792
_EP2 = _NS * _EPT
_Z0 = _NP * _W0 // _NS
_Z1 = _NP * _W1 // _NS
_ZC = 7168


def _adj_counts_kernel(edge_hbm, out_hbm, e_v, idx_v, ones_v, z_v, acc_sh,
                       sem):
    c = lax.axis_index("c")
    s = lax.axis_index("s")
    base = s * _EPT
    vw = 512 - 128 * c
    @pl.loop(0, _ZC, step=16)
    def _(i):
        z_v[pl.ds(i, 16)] = jnp.zeros((16,), jnp.float32)
    zdma = [pltpu.async_copy(z_v, acc_sh.at[pl.ds(s * _Z0 + q * _ZC, _ZC)],
                             sem) for q in range(4)]

    @pl.loop(0, 128, step=16)
    def _(i):
        ones_v[pl.ds(i, 16)] = jnp.ones((16,), jnp.float32)

    pltpu.sync_copy(edge_hbm.at[pl.ds(0, 2), pl.ds(base, _EPT)], e_v)

    lane = lax.iota(jnp.int32, 16)
    sent = _SENT * _W0 + s * 16 + lane
    for j in range(14):
        @pl.loop(0, 128, step=16)
        def _(k, j=j):
            e = j * 128 + k
            d16 = e_v[1, pl.ds(e, 16)]
            local = e_v[0, pl.ds(e, 16)] - 512 * c
            valid = (local >= 0) & (local < vw)
            idx_v[j, pl.ds(k, 16)] = jnp.where(valid, d16 * _W0 + local, sent)

    for h in zdma:
        h.wait()
    plsc.subcore_barrier()
    sdma = [pltpu.async_copy(ones_v, acc_sh.at[idx_v.at[j]], sem, add=True)
            for j in range(14)]
    for h in sdma:
        h.wait()
    plsc.subcore_barrier()
    pdma = [pltpu.async_copy(acc_sh.at[pl.ds((s * 56 + r) * _W0, _W0)],
                             out_hbm.at[c, s, r], sem) for r in range(56)]
    for h in pdma:
        h.wait()


@functools.cache
def _adj_counts():
    mesh = plsc.VectorSubcoreMesh(core_axis_name="c", subcore_axis_name="s")
    return pl.kernel(
        _adj_counts_kernel,
        out_type=jax.ShapeDtypeStruct((_NC, _NS, _NP // _NS, _W0),
                                      jnp.float32),
        mesh=mesh,
        scratch_types=[
            pltpu.VMEM((2, _EPT), jnp.int32),
            pltpu.VMEM((14, 128), jnp.int32),
            pltpu.VMEM((128,), jnp.float32),
            pltpu.VMEM((_ZC,), jnp.float32),
            pltpu.VMEM_SHARED((_NP * _W0,), jnp.float32),
            pltpu.SemaphoreType.DMA,
        ],
    )


def _dot(a, b):
    return jnp.dot(a, b, preferred_element_type=jnp.float32)


def _dot_t(a, b):
    return lax.dot_general(a, b, (((1,), (1,)), ((), ())),
                           preferred_element_type=jnp.float32)


def _tc_body(a_ref, f_ref, wm_ref, bm_ref, wd_ref, bd_ref,
             ws1_ref, wn1_ref, b1_ref, ws2_ref, wn2_ref, b2_ref, o_ref):
    f = f_ref[...]
    xm = _dot_t(f, wm_ref[...]) + bm_ref[...].reshape(1, _F)
    xd = _dot_t(f[:, :_N_DIS], wd_ref[...]) + bd_ref[...].reshape(1, _F)
    row = lax.broadcasted_iota(jnp.int32, (_N, _F), 0)
    x878 = jnp.where(row < _N_MIRNA, xm, xd)
    x = jnp.concatenate([x878, jnp.zeros((_NP - _N, _F), jnp.float32)], axis=0)
    p0 = jnp.reshape(a_ref[0], (_NP, _W0))
    p1 = jnp.reshape(a_ref[1], (_NP, _W0))
    a = jnp.concatenate([p0, p1[:, :_W1]], axis=1)
    deg = jnp.sum(a, axis=1, keepdims=True)
    inv = 1.0 / jnp.maximum(deg, 1.0)
    n1 = _dot(a, x) * inv
    h1 = jnp.maximum(_dot_t(x, ws1_ref[...]) + _dot_t(n1, wn1_ref[...])
                     + b1_ref[...].reshape(1, _F), 0.0)
    n2 = _dot(a, h1) * inv
    h2 = jnp.maximum(_dot_t(h1, ws2_ref[...]) + _dot_t(n2, wn2_ref[...])
                     + b2_ref[...].reshape(1, _F), 0.0)
    o_ref[...] = h2[:_N]


_tc = pl.pallas_call(
    _tc_body, out_shape=jax.ShapeDtypeStruct((_N, _F), jnp.float32))


def kernel(in_feat, edge_index, Wm, bm, Wd, bd, Ws1, Wn1, b1, Ws2, Wn2, b2):
    edge_p = jnp.pad(edge_index, ((0, 0), (0, _EP2 - _E)),
                     constant_values=_SENT)
    counts = _adj_counts()(edge_p)
    return _tc(counts, in_feat, Wm, bm, Wd, bd, Ws1, Wn1, b1, Ws2, Wn2, b2)


# speedup vs baseline: 1.0513x; 1.0513x over previous
import functools

import jax
import jax.numpy as jnp
from jax import lax
from jax.experimental import pallas as pl
from jax.experimental.pallas import tpu as pltpu
from jax.experimental.pallas import tpu_sc as plsc

_N_MIRNA = 495
_N_DIS = 383
_N = 878
_NP = 896
_F = 256
_E = 28096
_NC = 2
_NS = 16
_SENT = _NP - 1
_W0 = 512
_W1 = _NP - _W0
_EPT = 1792
_EP2 = _NS * _EPT
_Z0 = _NP * _W0 // _NS
_Z1 = _NP * _W1 // _NS
_ZC = 7168


def _adj_counts_kernel(edge_hbm, out_hbm, e_v, idx_v, ones_v, z_v, acc_sh,
                       sem):
    c = lax.axis_index("c")
    s = lax.axis_index("s")
    base = s * _EPT
    w = 512 - 128 * c
    @pl.loop(0, _ZC, step=16)
    def _(i):
        z_v[pl.ds(i, 16)] = jnp.zeros((16,), jnp.float32)
    zbase = s * (_Z0 - 7168 * c)
    zdma = [pltpu.async_copy(z_v, acc_sh.at[pl.ds(zbase + q * _ZC, _ZC)],
                             sem) for q in range(3)]

    @pl.loop(0, 128, step=16)
    def _(i):
        ones_v[pl.ds(i, 16)] = jnp.ones((16,), jnp.float32)

    @pl.when(c == 0)
    def _():
        pltpu.sync_copy(z_v, acc_sh.at[pl.ds(zbase + 3 * _ZC, _ZC)])

    pltpu.sync_copy(edge_hbm.at[pl.ds(0, 2), pl.ds(base, _EPT)], e_v)

    lane = lax.iota(jnp.int32, 16)
    sent = _SENT * w + s * 16 + lane
    for j in range(14):
        @pl.loop(0, 128, step=16)
        def _(k, j=j):
            e = j * 128 + k
            d16 = e_v[1, pl.ds(e, 16)]
            local = e_v[0, pl.ds(e, 16)] - 512 * c
            valid = (local >= 0) & (local < w)
            idx_v[j, pl.ds(k, 16)] = jnp.where(valid, d16 * w + local, sent)

    for h in zdma:
        h.wait()
    plsc.subcore_barrier()
    sdma = [pltpu.async_copy(ones_v, acc_sh.at[idx_v.at[j]], sem, add=True)
            for j in range(14)]
    for h in sdma:
        h.wait()
    plsc.subcore_barrier()
    @pl.when(c == 0)
    def _():
        pltpu.sync_copy(acc_sh.at[pl.ds(s * _Z0, _Z0)], out_hbm.at[0, s])

    @pl.when(c == 1)
    def _():
        pltpu.sync_copy(acc_sh.at[pl.ds(s * _Z1, _Z1)],
                        out_hbm.at[1, s, pl.ds(0, _Z1)])


@functools.cache
def _adj_counts():
    mesh = plsc.VectorSubcoreMesh(core_axis_name="c", subcore_axis_name="s")
    return pl.kernel(
        _adj_counts_kernel,
        out_type=jax.ShapeDtypeStruct((_NC, _NS, _Z0), jnp.float32),
        mesh=mesh,
        scratch_types=[
            pltpu.VMEM((2, _EPT), jnp.int32),
            pltpu.VMEM((14, 128), jnp.int32),
            pltpu.VMEM((128,), jnp.float32),
            pltpu.VMEM((_ZC,), jnp.float32),
            pltpu.VMEM_SHARED((_NP * _W0,), jnp.float32),
            pltpu.SemaphoreType.DMA,
        ],
    )


def _dot(a, b):
    return jnp.dot(a, b, preferred_element_type=jnp.float32)


def _dot_t(a, b):
    return lax.dot_general(a, b, (((1,), (1,)), ((), ())),
                           preferred_element_type=jnp.float32)


def _tc_body(a_ref, f_ref, wm_ref, bm_ref, wd_ref, bd_ref,
             ws1_ref, wn1_ref, b1_ref, ws2_ref, wn2_ref, b2_ref, o_ref):
    f = f_ref[...]
    xm = _dot_t(f, wm_ref[...]) + bm_ref[...].reshape(1, _F)
    xd = _dot_t(f[:, :_N_DIS], wd_ref[...]) + bd_ref[...].reshape(1, _F)
    row = lax.broadcasted_iota(jnp.int32, (_N, _F), 0)
    x878 = jnp.where(row < _N_MIRNA, xm, xd)
    x = jnp.concatenate([x878, jnp.zeros((_NP - _N, _F), jnp.float32)], axis=0)
    p0 = jnp.reshape(a_ref[0], (_NP, _W0))
    p1 = jnp.reshape(a_ref[1][:, :_Z1], (_NP, _W1))
    a = jnp.concatenate([p0, p1], axis=1)
    deg = jnp.sum(a, axis=1, keepdims=True)
    inv = 1.0 / jnp.maximum(deg, 1.0)
    n1 = _dot(a, x) * inv
    h1 = jnp.maximum(_dot_t(x, ws1_ref[...]) + _dot_t(n1, wn1_ref[...])
                     + b1_ref[...].reshape(1, _F), 0.0)
    n2 = _dot(a, h1) * inv
    h2 = jnp.maximum(_dot_t(h1, ws2_ref[...]) + _dot_t(n2, wn2_ref[...])
                     + b2_ref[...].reshape(1, _F), 0.0)
    o_ref[...] = h2[:_N]


_tc = pl.pallas_call(
    _tc_body, out_shape=jax.ShapeDtypeStruct((_N, _F), jnp.float32))


def kernel(in_feat, edge_index, Wm, bm, Wd, bd, Ws1, Wn1, b1, Ws2, Wn2, b2):
    edge_p = jnp.pad(edge_index, ((0, 0), (0, _EP2 - _E)),
                     constant_values=_SENT)
    counts = _adj_counts()(edge_p)
    return _tc(counts, in_feat, Wm, bm, Wd, bd, Ws1, Wn1, b1, Ws2, Wn2, b2)


# speedup vs baseline: 1.0899x; 1.0368x over previous
import functools

import jax
import jax.numpy as jnp
from jax import lax
from jax.experimental import pallas as pl
from jax.experimental.pallas import tpu as pltpu
from jax.experimental.pallas import tpu_sc as plsc

_N_MIRNA = 495
_N_DIS = 383
_N = 878
_NP = 896
_F = 256
_E = 28096
_NC = 2
_NS = 16
_SENT = _NP - 1
_W0 = 512
_W1 = _NP - _W0
_EPT = 1792
_EP2 = _NS * _EPT
_Z0 = _NP * _W0 // _NS
_Z1 = _NP * _W1 // _NS
_ZC = 7168


def _adj_counts_kernel(edge_hbm, out_hbm, e_v, idx_v, ones_v, z_v, acc_sh,
                       sem):
    c = lax.axis_index("c")
    s = lax.axis_index("s")
    base = s * _EPT
    w = 512 - 128 * c
    @pl.loop(0, _ZC, step=16)
    def _(i):
        z_v[pl.ds(i, 16)] = jnp.zeros((16,), jnp.float32)
    zbase = s * (_Z0 - 7168 * c)
    zdma = [pltpu.async_copy(z_v, acc_sh.at[pl.ds(zbase + q * _ZC, _ZC)],
                             sem) for q in range(3)]

    @pl.loop(0, 128, step=16)
    def _(i):
        ones_v[pl.ds(i, 16)] = jnp.ones((16,), jnp.float32)

    @pl.when(c == 0)
    def _():
        pltpu.sync_copy(z_v, acc_sh.at[pl.ds(zbase + 3 * _ZC, _ZC)])

    pltpu.sync_copy(edge_hbm.at[pl.ds(0, 2), pl.ds(base, _EPT)], e_v)

    lane = lax.iota(jnp.int32, 16)
    sent = _SENT * w + s * 16 + lane
    for j in range(14):
        @pl.loop(0, 128, step=16)
        def _(k, j=j):
            e = j * 128 + k
            d16 = e_v[1, pl.ds(e, 16)]
            local = e_v[0, pl.ds(e, 16)] - 512 * c
            valid = (local >= 0) & (local < w)
            idx_v[j, pl.ds(k, 16)] = jnp.where(valid, d16 * w + local, sent)

    for h in zdma:
        h.wait()
    plsc.subcore_barrier()
    sdma = [pltpu.async_copy(ones_v, acc_sh.at[idx_v.at[j]], sem, add=True)
            for j in range(14)]
    for h in sdma:
        h.wait()
    plsc.subcore_barrier()
    @pl.when(c == 0)
    def _():
        pltpu.sync_copy(acc_sh.at[pl.ds(s * _Z0, _Z0)], out_hbm.at[0, s])

    @pl.when(c == 1)
    def _():
        pltpu.sync_copy(acc_sh.at[pl.ds(s * _Z1, _Z1)],
                        out_hbm.at[1, s, pl.ds(0, _Z1)])


@functools.cache
def _adj_counts():
    mesh = plsc.VectorSubcoreMesh(core_axis_name="c", subcore_axis_name="s")
    return pl.kernel(
        _adj_counts_kernel,
        out_type=jax.ShapeDtypeStruct((_NC, _NS, _Z0), jnp.float32),
        mesh=mesh,
        scratch_types=[
            pltpu.VMEM((2, _EPT), jnp.int32),
            pltpu.VMEM((14, 128), jnp.int32),
            pltpu.VMEM((128,), jnp.float32),
            pltpu.VMEM((_ZC,), jnp.float32),
            pltpu.VMEM_SHARED((_NP * _W0,), jnp.float32),
            pltpu.SemaphoreType.DMA,
        ],
    )


def _dot(a, b):
    return jnp.dot(a, b, preferred_element_type=jnp.float32)


def _dot_t(a, b):
    return lax.dot_general(a, b, (((1,), (1,)), ((), ())),
                           preferred_element_type=jnp.float32)


def _fc_body(f_ref, wm_ref, bm_ref, wd_ref, bd_ref, x_ref):
    f = f_ref[...]
    xm = _dot_t(f, wm_ref[...]) + bm_ref[...].reshape(1, _F)
    xd = _dot_t(f[:, :_N_DIS], wd_ref[...]) + bd_ref[...].reshape(1, _F)
    row = lax.broadcasted_iota(jnp.int32, (_N, _F), 0)
    x878 = jnp.where(row < _N_MIRNA, xm, xd)
    x_ref[...] = jnp.concatenate(
        [x878, jnp.zeros((_NP - _N, _F), jnp.float32)], axis=0)


_fc = pl.pallas_call(
    _fc_body, out_shape=jax.ShapeDtypeStruct((_NP, _F), jnp.float32))


def _tc_body(a_ref, x_ref, ws1_ref, wn1_ref, b1_ref, ws2_ref, wn2_ref,
             b2_ref, o_ref):
    x = x_ref[...]
    p0 = jnp.reshape(a_ref[0], (_NP, _W0))
    p1 = jnp.reshape(a_ref[1][:, :_Z1], (_NP, _W1))
    a = jnp.concatenate([p0, p1], axis=1)
    deg = jnp.sum(a, axis=1, keepdims=True)
    inv = 1.0 / jnp.maximum(deg, 1.0)
    n1 = _dot(a, x) * inv
    h1 = jnp.maximum(_dot_t(x, ws1_ref[...]) + _dot_t(n1, wn1_ref[...])
                     + b1_ref[...].reshape(1, _F), 0.0)
    n2 = _dot(a, h1) * inv
    h2 = jnp.maximum(_dot_t(h1, ws2_ref[...]) + _dot_t(n2, wn2_ref[...])
                     + b2_ref[...].reshape(1, _F), 0.0)
    o_ref[...] = h2[:_N]


_tc = pl.pallas_call(
    _tc_body, out_shape=jax.ShapeDtypeStruct((_N, _F), jnp.float32))


def kernel(in_feat, edge_index, Wm, bm, Wd, bd, Ws1, Wn1, b1, Ws2, Wn2, b2):
    edge_p = jnp.pad(edge_index, ((0, 0), (0, _EP2 - _E)),
                     constant_values=_SENT)
    counts = _adj_counts()(edge_p)
    x = _fc(in_feat, Wm, bm, Wd, bd)
    return _tc(counts, x, Ws1, Wn1, b1, Ws2, Wn2, b2)
